# baseline (speedup 1.0000x reference)
import functools

import jax
import jax.numpy as jnp
from jax import lax
from jax.experimental import pallas as pl
from jax.experimental.pallas import tpu as pltpu
from jax.experimental.pallas import tpu_sc as plsc

NUM_NUMERICAL = 13
N_FIELDS = 26
FIELD_SIZE = 100000
EMB_DIM = 64
BATCH = 16384
N_OUT = N_FIELDS + 1
N_VOCAB = N_FIELDS * FIELD_SIZE

NC, NS = 2, 16
NW = NC * NS
BPW = BATCH // NW
CB = 16
NCHUNK = BPW // CB
LPC = CB * N_FIELDS
GU = 104
GPC = LPC // GU
IDXR = BATCH * N_FIELDS // GU

MLP_BT = 2048
TR_BT = 24576


def _mlp_body(x_ref, w1, b1, w2, b2, w3, b3, o_ref):
    h = jnp.dot(x_ref[...], w1[...], preferred_element_type=jnp.float32)
    h = jnp.maximum(h + b1[...], 0.0)
    h = jnp.dot(h, w2[...], preferred_element_type=jnp.float32)
    h = jnp.maximum(h + b2[...], 0.0)
    h = jnp.dot(h, w3[...], preferred_element_type=jnp.float32)
    o_ref[...] = jnp.maximum(h + b3[...], 0.0)


def _mlp(numerical_input, W1, b1, W2, b2, W3, b3):
    d1, d2, d3 = W1.shape[1], W2.shape[1], W3.shape[1]
    return pl.pallas_call(
        _mlp_body,
        grid=(BATCH // MLP_BT,),
        in_specs=[
            pl.BlockSpec((MLP_BT, NUM_NUMERICAL), lambda i: (i, 0)),
            pl.BlockSpec((NUM_NUMERICAL, d1), lambda i: (0, 0)),
            pl.BlockSpec((1, d1), lambda i: (0, 0)),
            pl.BlockSpec((d1, d2), lambda i: (0, 0)),
            pl.BlockSpec((1, d2), lambda i: (0, 0)),
            pl.BlockSpec((d2, d3), lambda i: (0, 0)),
            pl.BlockSpec((1, d3), lambda i: (0, 0)),
        ],
        out_specs=pl.BlockSpec((MLP_BT, d3), lambda i: (i, 0)),
        out_shape=jax.ShapeDtypeStruct((BATCH, d3), jnp.float32),
    )(numerical_input, W1, b1.reshape(1, -1), W2, b2.reshape(1, -1),
      W3, b3.reshape(1, -1))


def _transpose_body(t_ref, o_ref):
    r = lax.broadcasted_iota(jnp.int32, (EMB_DIM, EMB_DIM), 0)
    c = lax.broadcasted_iota(jnp.int32, (EMB_DIM, EMB_DIM), 1)
    eye = (r == c).astype(jnp.float32)
    o_ref[:, 0:EMB_DIM] = lax.dot_general(
        t_ref[...], eye, (((0,), (0,)), ((), ())),
        preferred_element_type=jnp.float32)


def _transpose_table(tableT):
    n = tableT.shape[1]
    return pl.pallas_call(
        _transpose_body,
        grid=(pl.cdiv(n, TR_BT),),
        in_specs=[pl.BlockSpec((EMB_DIM, TR_BT), lambda i: (0, i))],
        out_specs=pl.BlockSpec((TR_BT, 2 * EMB_DIM), lambda i: (i, 0)),
        out_shape=jax.ShapeDtypeStruct((n, 2 * EMB_DIM), jnp.float32),
    )(tableT)


def _idx_body(cat_ref, fidx_ref):
    r = lax.broadcasted_iota(jnp.int32, (IDXR, GU), 0)
    c = lax.broadcasted_iota(jnp.int32, (IDXR, GU), 1)
    p = r * GU + c
    f = p % N_FIELDS
    fidx_ref[:, 0:GU] = cat_ref[...] + f * FIELD_SIZE
    fidx_ref[:, GU:128] = jnp.zeros((IDXR, 128 - GU), jnp.int32)


def _idx_prep(catg):
    return pl.pallas_call(
        _idx_body,
        out_shape=jax.ShapeDtypeStruct((IDXR, 128), jnp.int32),
    )(catg)


@functools.partial(
    pl.kernel,
    out_type=jax.ShapeDtypeStruct((BATCH, N_OUT, EMB_DIM), jnp.float32),
    mesh=plsc.VectorSubcoreMesh(
        core_axis_name="c", subcore_axis_name="s",
        num_cores=NC, num_subcores=NS),
    compiler_params=pltpu.CompilerParams(use_tc_tiling_on_sc=False),
    scratch_types=[
        pltpu.VMEM((GPC, 128), jnp.int32),
        pltpu.VMEM((LPC, 2 * EMB_DIM), jnp.float32),
        pltpu.VMEM((CB, EMB_DIM), jnp.float32),
        pltpu.VMEM((CB, N_OUT, EMB_DIM), jnp.float32),
        pltpu.SemaphoreType.DMA,
    ],
)
def _sc_gather(fidx_hbm, mlp_hbm, table_hbm, out_hbm,
               idx_v, rows_v, mlp_v, blk_v, gsem):
    cid = lax.axis_index("c")
    sid = lax.axis_index("s")
    wid = sid * NC + cid

    def chunk_body(k, carry):
        irow = wid * (NCHUNK * GPC) + k * GPC
        b0 = wid * BPW + k * CB

        pltpu.sync_copy(fidx_hbm.at[pl.ds(irow, GPC)], idx_v)
        copies = []
        for s in range(GPC):
            copies.append(pltpu.async_copy(
                table_hbm.at[idx_v.at[s, pl.ds(0, GU)]],
                rows_v.at[pl.ds(s * GU, GU)], gsem))
        pltpu.sync_copy(mlp_hbm.at[pl.ds(b0, CB)], mlp_v)
        for cp in copies:
            cp.wait()

        def bbody(bl, bcarry):
            for s in range(EMB_DIM // 16):
                o = s * 16
                blk_v[bl, 0, pl.ds(o, 16)] = mlp_v[bl, pl.ds(o, 16)]
            for jj in range(N_FIELDS):
                for s in range(EMB_DIM // 16):
                    o = s * 16
                    blk_v[bl, jj + 1, pl.ds(o, 16)] = (
                        rows_v[bl * N_FIELDS + jj, pl.ds(o, 16)])
            return bcarry

        lax.fori_loop(0, CB, bbody, 0)

        pltpu.sync_copy(blk_v, out_hbm.at[pl.ds(b0, CB)])
        return carry

    lax.fori_loop(0, NCHUNK, chunk_body, 0)


def kernel(numerical_input, categorical_inputs, W1, b1, W2, b2, W3, b3, table):
    mlp_out = _mlp(numerical_input, W1, b1, W2, b2, W3, b3)
    tableP = _transpose_table(jnp.swapaxes(table, 0, 1))
    fidx = _idx_prep(categorical_inputs.reshape(IDXR, GU))
    return _sc_gather(fidx, mlp_out, tableP)


# speedup vs baseline: 1.0295x; 1.0295x over previous
import functools

import jax
import jax.numpy as jnp
from jax import lax
from jax.experimental import pallas as pl
from jax.experimental.pallas import tpu as pltpu
from jax.experimental.pallas import tpu_sc as plsc

NUM_NUMERICAL = 13
N_FIELDS = 26
FIELD_SIZE = 100000
EMB_DIM = 64
BATCH = 16384
N_OUT = N_FIELDS + 1
N_VOCAB = N_FIELDS * FIELD_SIZE

NC, NS = 2, 16
NW = NC * NS
BPW = BATCH // NW
CB = 8
NCHUNK = BPW // CB
LPC = CB * N_FIELDS
GU = 104
GPC = LPC // GU
IDXR = BATCH * N_FIELDS // GU

MLP_BT = 2048
TR_BT = 32768


def _mlp_body(x_ref, w1, b1, w2, b2, w3, b3, o_ref):
    h = jnp.dot(x_ref[...], w1[...], preferred_element_type=jnp.float32)
    h = jnp.maximum(h + b1[...], 0.0)
    h = jnp.dot(h, w2[...], preferred_element_type=jnp.float32)
    h = jnp.maximum(h + b2[...], 0.0)
    h = jnp.dot(h, w3[...], preferred_element_type=jnp.float32)
    o_ref[...] = jnp.maximum(h + b3[...], 0.0)


def _mlp(numerical_input, W1, b1, W2, b2, W3, b3):
    d1, d2, d3 = W1.shape[1], W2.shape[1], W3.shape[1]
    return pl.pallas_call(
        _mlp_body,
        grid=(BATCH // MLP_BT,),
        in_specs=[
            pl.BlockSpec((MLP_BT, NUM_NUMERICAL), lambda i: (i, 0)),
            pl.BlockSpec((NUM_NUMERICAL, d1), lambda i: (0, 0)),
            pl.BlockSpec((1, d1), lambda i: (0, 0)),
            pl.BlockSpec((d1, d2), lambda i: (0, 0)),
            pl.BlockSpec((1, d2), lambda i: (0, 0)),
            pl.BlockSpec((d2, d3), lambda i: (0, 0)),
            pl.BlockSpec((1, d3), lambda i: (0, 0)),
        ],
        out_specs=pl.BlockSpec((MLP_BT, d3), lambda i: (i, 0)),
        out_shape=jax.ShapeDtypeStruct((BATCH, d3), jnp.float32),
    )(numerical_input, W1, b1.reshape(1, -1), W2, b2.reshape(1, -1),
      W3, b3.reshape(1, -1))


def _transpose_body(t_ref, o_ref):
    r = lax.broadcasted_iota(jnp.int32, (EMB_DIM, EMB_DIM), 0)
    c = lax.broadcasted_iota(jnp.int32, (EMB_DIM, EMB_DIM), 1)
    eye = (r == c).astype(jnp.float32)
    o_ref[:, 0:EMB_DIM] = lax.dot_general(
        t_ref[...], eye, (((0,), (0,)), ((), ())),
        preferred_element_type=jnp.float32)


def _transpose_table(tableT):
    n = tableT.shape[1]
    return pl.pallas_call(
        _transpose_body,
        grid=(pl.cdiv(n, TR_BT),),
        in_specs=[pl.BlockSpec((EMB_DIM, TR_BT), lambda i: (0, i))],
        out_specs=pl.BlockSpec((TR_BT, 2 * EMB_DIM), lambda i: (i, 0)),
        out_shape=jax.ShapeDtypeStruct((n, 2 * EMB_DIM), jnp.float32),
    )(tableT)


def _idx_body(cat_ref, fidx_ref):
    r = lax.broadcasted_iota(jnp.int32, (IDXR, GU), 0)
    c = lax.broadcasted_iota(jnp.int32, (IDXR, GU), 1)
    p = r * GU + c
    f = p % N_FIELDS
    fidx_ref[:, 0:GU] = cat_ref[...] + f * FIELD_SIZE
    fidx_ref[:, GU:128] = jnp.zeros((IDXR, 128 - GU), jnp.int32)


def _idx_prep(catg):
    return pl.pallas_call(
        _idx_body,
        out_shape=jax.ShapeDtypeStruct((IDXR, 128), jnp.int32),
    )(catg)


@functools.partial(
    pl.kernel,
    out_type=jax.ShapeDtypeStruct((BATCH, N_OUT, EMB_DIM), jnp.float32),
    mesh=plsc.VectorSubcoreMesh(
        core_axis_name="c", subcore_axis_name="s",
        num_cores=NC, num_subcores=NS),
    compiler_params=pltpu.CompilerParams(use_tc_tiling_on_sc=False),
    scratch_types=[
        pltpu.VMEM((GPC, 128), jnp.int32),
        pltpu.VMEM((GPC, 128), jnp.int32),
        pltpu.VMEM((LPC, 2 * EMB_DIM), jnp.float32),
        pltpu.VMEM((LPC, 2 * EMB_DIM), jnp.float32),
        pltpu.VMEM((CB, EMB_DIM), jnp.float32),
        pltpu.VMEM((CB, N_OUT, EMB_DIM), jnp.float32),
        pltpu.SemaphoreType.DMA,
        pltpu.SemaphoreType.DMA,
    ],
)
def _sc_gather(fidx_hbm, mlp_hbm, table_hbm, out_hbm,
               idx0_v, idx1_v, rows0_v, rows1_v, mlp_v, blk_v, sem0, sem1):
    cid = lax.axis_index("c")
    sid = lax.axis_index("s")
    wid = sid * NC + cid

    idx_b = (idx0_v, idx1_v)
    rows_b = (rows0_v, rows1_v)
    sem_b = (sem0, sem1)

    def fire(k, ph):
        irow = wid * (NCHUNK * GPC) + k * GPC
        pltpu.sync_copy(fidx_hbm.at[pl.ds(irow, GPC)], idx_b[ph])
        for s in range(GPC):
            pltpu.async_copy(
                table_hbm.at[idx_b[ph].at[s, pl.ds(0, GU)]],
                rows_b[ph].at[pl.ds(s * GU, GU)], sem_b[ph])

    def drain(ph):
        for s in range(GPC):
            pltpu.make_async_copy(
                table_hbm.at[pl.ds(0, GU)],
                rows_b[ph].at[pl.ds(s * GU, GU)], sem_b[ph]).wait()

    def process(k, ph):
        b0 = wid * BPW + k * CB
        pltpu.sync_copy(mlp_hbm.at[pl.ds(b0, CB)], mlp_v)
        drain(ph)

        def bbody(bl, bcarry):
            for s in range(EMB_DIM // 16):
                o = s * 16
                blk_v[bl, 0, pl.ds(o, 16)] = mlp_v[bl, pl.ds(o, 16)]
            for jj in range(N_FIELDS):
                for s in range(EMB_DIM // 16):
                    o = s * 16
                    blk_v[bl, jj + 1, pl.ds(o, 16)] = (
                        rows_b[ph][bl * N_FIELDS + jj, pl.ds(o, 16)])
            return bcarry

        lax.fori_loop(0, CB, bbody, 0)
        pltpu.sync_copy(blk_v, out_hbm.at[pl.ds(b0, CB)])

    fire(0, 0)

    def pair_body(k2, carry):
        for ph in range(2):
            k = k2 * 2 + ph
            fire(lax.rem(k + 1, NCHUNK), 1 - ph)
            process(k, ph)
        return carry

    lax.fori_loop(0, NCHUNK // 2, pair_body, 0)
    drain(0)


def kernel(numerical_input, categorical_inputs, W1, b1, W2, b2, W3, b3, table):
    mlp_out = _mlp(numerical_input, W1, b1, W2, b2, W3, b3)
    tableP = _transpose_table(jnp.swapaxes(table, 0, 1))
    fidx = _idx_prep(categorical_inputs.reshape(IDXR, GU))
    return _sc_gather(fidx, mlp_out, tableP)


# speedup vs baseline: 1.0938x; 1.0624x over previous
import functools

import jax
import jax.numpy as jnp
from jax import lax
from jax.experimental import pallas as pl
from jax.experimental.pallas import tpu as pltpu
from jax.experimental.pallas import tpu_sc as plsc

NUM_NUMERICAL = 13
N_FIELDS = 26
FIELD_SIZE = 100000
EMB_DIM = 64
BATCH = 16384
N_OUT = N_FIELDS + 1
N_VOCAB = N_FIELDS * FIELD_SIZE

NC, NS = 2, 16
NW = NC * NS
BPW = BATCH // NW
CB = 8
NCHUNK = BPW // CB
LPC = CB * N_FIELDS
GU = 104
GPC = LPC // GU
IDXR = BATCH * N_FIELDS // GU

MLP_BT = 2048
TR_BT = 32768


def _mlp_body(x_ref, w1, b1, w2, b2, w3, b3, o_ref):
    h = jnp.dot(x_ref[...], w1[...], preferred_element_type=jnp.float32)
    h = jnp.maximum(h + b1[...], 0.0)
    h = jnp.dot(h, w2[...], preferred_element_type=jnp.float32)
    h = jnp.maximum(h + b2[...], 0.0)
    h = jnp.dot(h, w3[...], preferred_element_type=jnp.float32)
    o_ref[...] = jnp.maximum(h + b3[...], 0.0)


def _mlp(numerical_input, W1, b1, W2, b2, W3, b3):
    d1, d2, d3 = W1.shape[1], W2.shape[1], W3.shape[1]
    return pl.pallas_call(
        _mlp_body,
        grid=(BATCH // MLP_BT,),
        in_specs=[
            pl.BlockSpec((MLP_BT, NUM_NUMERICAL), lambda i: (i, 0)),
            pl.BlockSpec((NUM_NUMERICAL, d1), lambda i: (0, 0)),
            pl.BlockSpec((1, d1), lambda i: (0, 0)),
            pl.BlockSpec((d1, d2), lambda i: (0, 0)),
            pl.BlockSpec((1, d2), lambda i: (0, 0)),
            pl.BlockSpec((d2, d3), lambda i: (0, 0)),
            pl.BlockSpec((1, d3), lambda i: (0, 0)),
        ],
        out_specs=pl.BlockSpec((MLP_BT, d3), lambda i: (i, 0)),
        out_shape=jax.ShapeDtypeStruct((BATCH, d3), jnp.float32),
    )(numerical_input, W1, b1.reshape(1, -1), W2, b2.reshape(1, -1),
      W3, b3.reshape(1, -1))


def _transpose_body(t_ref, o_ref):
    r = lax.broadcasted_iota(jnp.int32, (EMB_DIM, EMB_DIM), 0)
    c = lax.broadcasted_iota(jnp.int32, (EMB_DIM, EMB_DIM), 1)
    eye = (r == c).astype(jnp.float32)
    o_ref[:, 0:EMB_DIM] = lax.dot_general(
        t_ref[...], eye, (((0,), (0,)), ((), ())),
        preferred_element_type=jnp.float32)


def _transpose_table(tableT):
    n = tableT.shape[1]
    return pl.pallas_call(
        _transpose_body,
        grid=(pl.cdiv(n, TR_BT),),
        in_specs=[pl.BlockSpec((EMB_DIM, TR_BT), lambda i: (0, i))],
        out_specs=pl.BlockSpec((TR_BT, 2 * EMB_DIM), lambda i: (i, 0)),
        out_shape=jax.ShapeDtypeStruct((n, 2 * EMB_DIM), jnp.float32),
    )(tableT)


def _idx_body(cat_ref, fidx_ref):
    r = lax.broadcasted_iota(jnp.int32, (IDXR, GU), 0)
    c = lax.broadcasted_iota(jnp.int32, (IDXR, GU), 1)
    p = r * GU + c
    f = p % N_FIELDS
    fidx_ref[:, 0:GU] = cat_ref[...] + f * FIELD_SIZE
    fidx_ref[:, GU:128] = jnp.zeros((IDXR, 128 - GU), jnp.int32)


def _idx_prep(catg):
    return pl.pallas_call(
        _idx_body,
        out_shape=jax.ShapeDtypeStruct((IDXR, 128), jnp.int32),
    )(catg)


@functools.partial(
    pl.kernel,
    out_type=jax.ShapeDtypeStruct((BATCH, N_OUT, EMB_DIM), jnp.float32),
    mesh=plsc.VectorSubcoreMesh(
        core_axis_name="c", subcore_axis_name="s",
        num_cores=NC, num_subcores=NS),
    compiler_params=pltpu.CompilerParams(use_tc_tiling_on_sc=False),
    scratch_types=[
        pltpu.VMEM((GPC, 128), jnp.int32),
        pltpu.VMEM((GPC, 128), jnp.int32),
        pltpu.VMEM((LPC, 2 * EMB_DIM), jnp.float32),
        pltpu.VMEM((LPC, 2 * EMB_DIM), jnp.float32),
        pltpu.VMEM((CB, EMB_DIM), jnp.float32),
        pltpu.VMEM((CB, EMB_DIM), jnp.float32),
        pltpu.VMEM((CB, N_OUT, EMB_DIM), jnp.float32),
        pltpu.VMEM((CB, N_OUT, EMB_DIM), jnp.float32),
        pltpu.SemaphoreType.DMA,
        pltpu.SemaphoreType.DMA,
        pltpu.SemaphoreType.DMA,
    ],
)
def _sc_gather(fidx_hbm, mlp_hbm, table_hbm, out_hbm,
               idx0_v, idx1_v, rows0_v, rows1_v, mlp0_v, mlp1_v,
               blk0_v, blk1_v, sem0, sem1, osem):
    cid = lax.axis_index("c")
    sid = lax.axis_index("s")
    wid = sid * NC + cid

    idx_b = (idx0_v, idx1_v)
    rows_b = (rows0_v, rows1_v)
    mlp_b = (mlp0_v, mlp1_v)
    blk_b = (blk0_v, blk1_v)
    sem_b = (sem0, sem1)

    def fire(k, ph):
        irow = wid * (NCHUNK * GPC) + k * GPC
        b0 = wid * BPW + k * CB
        pltpu.sync_copy(fidx_hbm.at[pl.ds(irow, GPC)], idx_b[ph])
        for s in range(GPC):
            pltpu.async_copy(
                table_hbm.at[idx_b[ph].at[s, pl.ds(0, GU)]],
                rows_b[ph].at[pl.ds(s * GU, GU)], sem_b[ph])
        pltpu.async_copy(mlp_hbm.at[pl.ds(b0, CB)], mlp_b[ph], sem_b[ph])

    def drain(ph):
        for s in range(GPC):
            pltpu.make_async_copy(
                table_hbm.at[pl.ds(0, GU)],
                rows_b[ph].at[pl.ds(s * GU, GU)], sem_b[ph]).wait()
        pltpu.make_async_copy(
            mlp_hbm.at[pl.ds(0, CB)], mlp_b[ph], sem_b[ph]).wait()

    def process(k2, k, ph):
        b0 = wid * BPW + k * CB
        drain(ph)

        @pl.when(k2 > 0)
        def _():
            pltpu.make_async_copy(
                out_hbm.at[pl.ds(0, CB)], blk_b[ph], osem).wait()

        def bbody(bl, bcarry):
            for s in range(EMB_DIM // 16):
                o = s * 16
                blk_b[ph][bl, 0, pl.ds(o, 16)] = mlp_b[ph][bl, pl.ds(o, 16)]
            for jj in range(N_FIELDS):
                for s in range(EMB_DIM // 16):
                    o = s * 16
                    blk_b[ph][bl, jj + 1, pl.ds(o, 16)] = (
                        rows_b[ph][bl * N_FIELDS + jj, pl.ds(o, 16)])
            return bcarry

        lax.fori_loop(0, CB, bbody, 0)
        pltpu.async_copy(blk_b[ph], out_hbm.at[pl.ds(b0, CB)], osem)

    fire(0, 0)

    def pair_body(k2, carry):
        for ph in range(2):
            k = k2 * 2 + ph
            fire(lax.rem(k + 1, NCHUNK), 1 - ph)
            process(k2, k, ph)
        return carry

    lax.fori_loop(0, NCHUNK // 2, pair_body, 0)
    drain(0)
    pltpu.make_async_copy(out_hbm.at[pl.ds(0, CB)], blk_b[0], osem).wait()
    pltpu.make_async_copy(out_hbm.at[pl.ds(0, CB)], blk_b[1], osem).wait()


def kernel(numerical_input, categorical_inputs, W1, b1, W2, b2, W3, b3, table):
    mlp_out = _mlp(numerical_input, W1, b1, W2, b2, W3, b3)
    tableP = _transpose_table(jnp.swapaxes(table, 0, 1))
    fidx = _idx_prep(categorical_inputs.reshape(IDXR, GU))
    return _sc_gather(fidx, mlp_out, tableP)


# speedup vs baseline: 1.1840x; 1.0824x over previous
import functools

import jax
import jax.numpy as jnp
from jax import lax
from jax.experimental import pallas as pl
from jax.experimental.pallas import tpu as pltpu
from jax.experimental.pallas import tpu_sc as plsc

NUM_NUMERICAL = 13
N_FIELDS = 26
FIELD_SIZE = 100000
EMB_DIM = 64
BATCH = 16384
N_OUT = N_FIELDS + 1
N_VOCAB = N_FIELDS * FIELD_SIZE

NC, NS = 2, 16
NW = NC * NS
BPW = BATCH // NW
CB = 8
NCHUNK = BPW // CB
LPC = CB * N_FIELDS
GU = 104
GPC = LPC // GU
IDXR = BATCH * N_FIELDS // GU

MLP_BT = 2048
TR_BT = 32768


def _mlp_body(x_ref, w1, b1, w2, b2, w3, b3, o_ref):
    h = jnp.dot(x_ref[...], w1[...], preferred_element_type=jnp.float32)
    h = jnp.maximum(h + b1[...], 0.0)
    h = jnp.dot(h, w2[...], preferred_element_type=jnp.float32)
    h = jnp.maximum(h + b2[...], 0.0)
    h = jnp.dot(h, w3[...], preferred_element_type=jnp.float32)
    o_ref[...] = jnp.maximum(h + b3[...], 0.0)


def _mlp(numerical_input, W1, b1, W2, b2, W3, b3):
    d1, d2, d3 = W1.shape[1], W2.shape[1], W3.shape[1]
    return pl.pallas_call(
        _mlp_body,
        grid=(BATCH // MLP_BT,),
        in_specs=[
            pl.BlockSpec((MLP_BT, NUM_NUMERICAL), lambda i: (i, 0)),
            pl.BlockSpec((NUM_NUMERICAL, d1), lambda i: (0, 0)),
            pl.BlockSpec((1, d1), lambda i: (0, 0)),
            pl.BlockSpec((d1, d2), lambda i: (0, 0)),
            pl.BlockSpec((1, d2), lambda i: (0, 0)),
            pl.BlockSpec((d2, d3), lambda i: (0, 0)),
            pl.BlockSpec((1, d3), lambda i: (0, 0)),
        ],
        out_specs=pl.BlockSpec((MLP_BT, d3), lambda i: (i, 0)),
        out_shape=jax.ShapeDtypeStruct((BATCH, d3), jnp.float32),
    )(numerical_input, W1, b1.reshape(1, -1), W2, b2.reshape(1, -1),
      W3, b3.reshape(1, -1))


def _transpose_body(t_ref, o_ref):
    r = lax.broadcasted_iota(jnp.int32, (EMB_DIM, EMB_DIM), 0)
    c = lax.broadcasted_iota(jnp.int32, (EMB_DIM, EMB_DIM), 1)
    eye = (r == c).astype(jnp.float32)
    o_ref[:, 0:EMB_DIM] = lax.dot_general(
        t_ref[...], eye, (((0,), (0,)), ((), ())),
        preferred_element_type=jnp.float32)


def _transpose_table(tableT):
    n = tableT.shape[1]
    return pl.pallas_call(
        _transpose_body,
        grid=(pl.cdiv(n, TR_BT),),
        in_specs=[pl.BlockSpec((EMB_DIM, TR_BT), lambda i: (0, i))],
        out_specs=pl.BlockSpec((TR_BT, 2 * EMB_DIM), lambda i: (i, 0)),
        out_shape=jax.ShapeDtypeStruct((n, 2 * EMB_DIM), jnp.float32),
    )(tableT)


def _idx_body(cat_ref, fidx_ref):
    r = lax.broadcasted_iota(jnp.int32, (IDXR, GU), 0)
    c = lax.broadcasted_iota(jnp.int32, (IDXR, GU), 1)
    p = r * GU + c
    f = p % N_FIELDS
    fidx_ref[:, 0:GU] = cat_ref[...] + f * FIELD_SIZE
    fidx_ref[:, GU:128] = jnp.zeros((IDXR, 128 - GU), jnp.int32)


def _idx_prep(catg):
    return pl.pallas_call(
        _idx_body,
        out_shape=jax.ShapeDtypeStruct((IDXR, 128), jnp.int32),
    )(catg)


@functools.partial(
    pl.kernel,
    out_type=jax.ShapeDtypeStruct((BATCH, N_OUT, EMB_DIM), jnp.float32),
    mesh=plsc.VectorSubcoreMesh(
        core_axis_name="c", subcore_axis_name="s",
        num_cores=NC, num_subcores=NS),
    compiler_params=pltpu.CompilerParams(use_tc_tiling_on_sc=False),
    scratch_types=[
        pltpu.VMEM((GPC, 128), jnp.int32),
        pltpu.VMEM((GPC, 128), jnp.int32),
        pltpu.VMEM((LPC, 2 * EMB_DIM), jnp.float32),
        pltpu.VMEM((LPC, 2 * EMB_DIM), jnp.float32),
        pltpu.VMEM((CB, EMB_DIM), jnp.float32),
        pltpu.VMEM((CB, EMB_DIM), jnp.float32),
        pltpu.VMEM((CB, N_OUT, EMB_DIM), jnp.float32),
        pltpu.VMEM((CB, N_OUT, EMB_DIM), jnp.float32),
        pltpu.SemaphoreType.DMA,
        pltpu.SemaphoreType.DMA,
        pltpu.SemaphoreType.DMA,
    ],
)
def _sc_gather(fidx_hbm, mlp_hbm, table_hbm, out_hbm,
               idx0_v, idx1_v, rows0_v, rows1_v, mlp0_v, mlp1_v,
               blk0_v, blk1_v, sem0, sem1, osem):
    cid = lax.axis_index("c")
    sid = lax.axis_index("s")
    wid = sid * NC + cid

    idx_b = (idx0_v, idx1_v)
    rows_b = (rows0_v, rows1_v)
    mlp_b = (mlp0_v, mlp1_v)
    blk_b = (blk0_v, blk1_v)
    sem_b = (sem0, sem1)

    def fire(k, ph):
        irow = wid * (NCHUNK * GPC) + k * GPC
        b0 = wid * BPW + k * CB
        pltpu.sync_copy(fidx_hbm.at[pl.ds(irow, GPC)], idx_b[ph])
        for s in range(GPC):
            pltpu.async_copy(
                table_hbm.at[idx_b[ph].at[s, pl.ds(0, GU)]],
                rows_b[ph].at[pl.ds(s * GU, GU)], sem_b[ph])
        pltpu.async_copy(mlp_hbm.at[pl.ds(b0, CB)], mlp_b[ph], sem_b[ph])

    def drain(ph):
        for s in range(GPC):
            pltpu.make_async_copy(
                table_hbm.at[pl.ds(0, GU)],
                rows_b[ph].at[pl.ds(s * GU, GU)], sem_b[ph]).wait()
        pltpu.make_async_copy(
            mlp_hbm.at[pl.ds(0, CB)], mlp_b[ph], sem_b[ph]).wait()

    def process(k2, k, ph):
        b0 = wid * BPW + k * CB
        drain(ph)

        @pl.when(k2 > 0)
        def _():
            pltpu.make_async_copy(
                out_hbm.at[pl.ds(0, CB)], blk_b[ph], osem).wait()

        for bl in range(CB):
            for s in range(EMB_DIM // 16):
                o = s * 16
                blk_b[ph][bl, 0, pl.ds(o, 16)] = mlp_b[ph][bl, pl.ds(o, 16)]
            for jj in range(N_FIELDS):
                for s in range(EMB_DIM // 16):
                    o = s * 16
                    blk_b[ph][bl, jj + 1, pl.ds(o, 16)] = (
                        rows_b[ph][bl * N_FIELDS + jj, pl.ds(o, 16)])
        pltpu.async_copy(blk_b[ph], out_hbm.at[pl.ds(b0, CB)], osem)

    fire(0, 0)

    def pair_body(k2, carry):
        for ph in range(2):
            k = k2 * 2 + ph
            fire(lax.rem(k + 1, NCHUNK), 1 - ph)
            process(k2, k, ph)
        return carry

    lax.fori_loop(0, NCHUNK // 2, pair_body, 0)
    drain(0)
    pltpu.make_async_copy(out_hbm.at[pl.ds(0, CB)], blk_b[0], osem).wait()
    pltpu.make_async_copy(out_hbm.at[pl.ds(0, CB)], blk_b[1], osem).wait()


def kernel(numerical_input, categorical_inputs, W1, b1, W2, b2, W3, b3, table):
    tableP = _transpose_table(jnp.swapaxes(table, 0, 1))
    mlp_out = _mlp(numerical_input, W1, b1, W2, b2, W3, b3)
    fidx = _idx_prep(categorical_inputs.reshape(IDXR, GU))
    return _sc_gather(fidx, mlp_out, tableP)


# speedup vs baseline: 1.1961x; 1.0102x over previous
import functools

import jax
import jax.numpy as jnp
from jax import lax
from jax.experimental import pallas as pl
from jax.experimental.pallas import tpu as pltpu
from jax.experimental.pallas import tpu_sc as plsc

NUM_NUMERICAL = 13
N_FIELDS = 26
FIELD_SIZE = 100000
EMB_DIM = 64
BATCH = 16384
N_OUT = N_FIELDS + 1
N_VOCAB = N_FIELDS * FIELD_SIZE

NC, NS = 2, 16
NW = NC * NS
BPW = BATCH // NW
CB = 8
NCHUNK = BPW // CB
LPC = CB * N_FIELDS
GU = 104
GPC = LPC // GU
IDXR = BATCH * N_FIELDS // GU

MLP_BT = 2048
TR_BT = 32768


def _mlp_body(x_ref, w1, b1, w2, b2, w3, b3, o_ref):
    h = jnp.dot(x_ref[...], w1[...], preferred_element_type=jnp.float32)
    h = jnp.maximum(h + b1[...], 0.0)
    h = jnp.dot(h, w2[...], preferred_element_type=jnp.float32)
    h = jnp.maximum(h + b2[...], 0.0)
    h = jnp.dot(h, w3[...], preferred_element_type=jnp.float32)
    o_ref[...] = jnp.maximum(h + b3[...], 0.0)


def _mlp(numerical_input, W1, b1, W2, b2, W3, b3):
    d1, d2, d3 = W1.shape[1], W2.shape[1], W3.shape[1]
    return pl.pallas_call(
        _mlp_body,
        grid=(BATCH // MLP_BT,),
        in_specs=[
            pl.BlockSpec((MLP_BT, NUM_NUMERICAL), lambda i: (i, 0)),
            pl.BlockSpec((NUM_NUMERICAL, d1), lambda i: (0, 0)),
            pl.BlockSpec((1, d1), lambda i: (0, 0)),
            pl.BlockSpec((d1, d2), lambda i: (0, 0)),
            pl.BlockSpec((1, d2), lambda i: (0, 0)),
            pl.BlockSpec((d2, d3), lambda i: (0, 0)),
            pl.BlockSpec((1, d3), lambda i: (0, 0)),
        ],
        out_specs=pl.BlockSpec((MLP_BT, d3), lambda i: (i, 0)),
        out_shape=jax.ShapeDtypeStruct((BATCH, d3), jnp.float32),
    )(numerical_input, W1, b1.reshape(1, -1), W2, b2.reshape(1, -1),
      W3, b3.reshape(1, -1))


def _transpose_body(t_ref, o_ref):
    r = lax.broadcasted_iota(jnp.int32, (EMB_DIM, EMB_DIM), 0)
    c = lax.broadcasted_iota(jnp.int32, (EMB_DIM, EMB_DIM), 1)
    eye = (r == c).astype(jnp.float32)
    o_ref[:, 0:EMB_DIM] = lax.dot_general(
        t_ref[...], eye, (((0,), (0,)), ((), ())),
        preferred_element_type=jnp.float32)


def _transpose_table(tableT):
    n = tableT.shape[1]
    return pl.pallas_call(
        _transpose_body,
        grid=(pl.cdiv(n, TR_BT),),
        in_specs=[pl.BlockSpec((EMB_DIM, TR_BT), lambda i: (0, i))],
        out_specs=pl.BlockSpec((TR_BT, 2 * EMB_DIM), lambda i: (i, 0)),
        out_shape=jax.ShapeDtypeStruct((n, 2 * EMB_DIM), jnp.float32),
    )(tableT)


def _idx_body(cat_ref, fidx_ref):
    r = lax.broadcasted_iota(jnp.int32, (IDXR, GU), 0)
    c = lax.broadcasted_iota(jnp.int32, (IDXR, GU), 1)
    p = r * GU + c
    f = p % N_FIELDS
    fidx_ref[:, 0:GU] = cat_ref[...] + f * FIELD_SIZE
    fidx_ref[:, GU:128] = jnp.zeros((IDXR, 128 - GU), jnp.int32)


def _idx_prep(catg):
    return pl.pallas_call(
        _idx_body,
        out_shape=jax.ShapeDtypeStruct((IDXR, 128), jnp.int32),
    )(catg)


@functools.partial(
    pl.kernel,
    out_type=jax.ShapeDtypeStruct((BATCH, N_OUT, EMB_DIM), jnp.float32),
    mesh=plsc.VectorSubcoreMesh(
        core_axis_name="c", subcore_axis_name="s",
        num_cores=NC, num_subcores=NS),
    compiler_params=pltpu.CompilerParams(use_tc_tiling_on_sc=False),
    scratch_types=[
        pltpu.VMEM((NCHUNK * GPC, 128), jnp.int32),
        pltpu.VMEM((LPC, 2 * EMB_DIM), jnp.float32),
        pltpu.VMEM((LPC, 2 * EMB_DIM), jnp.float32),
        pltpu.VMEM((CB, EMB_DIM), jnp.float32),
        pltpu.VMEM((CB, EMB_DIM), jnp.float32),
        pltpu.VMEM((CB, N_OUT, EMB_DIM), jnp.float32),
        pltpu.VMEM((CB, N_OUT, EMB_DIM), jnp.float32),
        pltpu.SemaphoreType.DMA,
        pltpu.SemaphoreType.DMA,
        pltpu.SemaphoreType.DMA,
    ],
)
def _sc_gather(fidx_hbm, mlp_hbm, table_hbm, out_hbm,
               idx_v, rows0_v, rows1_v, mlp0_v, mlp1_v,
               blk0_v, blk1_v, sem0, sem1, osem):
    cid = lax.axis_index("c")
    sid = lax.axis_index("s")
    wid = sid * NC + cid

    rows_b = (rows0_v, rows1_v)
    mlp_b = (mlp0_v, mlp1_v)
    blk_b = (blk0_v, blk1_v)
    sem_b = (sem0, sem1)

    pltpu.sync_copy(
        fidx_hbm.at[pl.ds(wid * (NCHUNK * GPC), NCHUNK * GPC)], idx_v)

    def fire(k, ph):
        b0 = wid * BPW + k * CB
        for s in range(GPC):
            pltpu.async_copy(
                table_hbm.at[idx_v.at[k * GPC + s, pl.ds(0, GU)]],
                rows_b[ph].at[pl.ds(s * GU, GU)], sem_b[ph])
        pltpu.async_copy(mlp_hbm.at[pl.ds(b0, CB)], mlp_b[ph], sem_b[ph])

    def drain(ph):
        for s in range(GPC):
            pltpu.make_async_copy(
                table_hbm.at[pl.ds(0, GU)],
                rows_b[ph].at[pl.ds(s * GU, GU)], sem_b[ph]).wait()
        pltpu.make_async_copy(
            mlp_hbm.at[pl.ds(0, CB)], mlp_b[ph], sem_b[ph]).wait()

    def process(k2, k, ph):
        b0 = wid * BPW + k * CB
        drain(ph)

        @pl.when(k2 > 0)
        def _():
            pltpu.make_async_copy(
                out_hbm.at[pl.ds(0, CB)], blk_b[ph], osem).wait()

        for bl in range(CB):
            for s in range(EMB_DIM // 16):
                o = s * 16
                blk_b[ph][bl, 0, pl.ds(o, 16)] = mlp_b[ph][bl, pl.ds(o, 16)]
            for jj in range(N_FIELDS):
                for s in range(EMB_DIM // 16):
                    o = s * 16
                    blk_b[ph][bl, jj + 1, pl.ds(o, 16)] = (
                        rows_b[ph][bl * N_FIELDS + jj, pl.ds(o, 16)])
        pltpu.async_copy(blk_b[ph], out_hbm.at[pl.ds(b0, CB)], osem)

    fire(0, 0)

    def pair_body(k2, carry):
        for ph in range(2):
            k = k2 * 2 + ph
            fire(lax.rem(k + 1, NCHUNK), 1 - ph)
            process(k2, k, ph)
        return carry

    lax.fori_loop(0, NCHUNK // 2, pair_body, 0)
    drain(0)
    pltpu.make_async_copy(out_hbm.at[pl.ds(0, CB)], blk_b[0], osem).wait()
    pltpu.make_async_copy(out_hbm.at[pl.ds(0, CB)], blk_b[1], osem).wait()


def kernel(numerical_input, categorical_inputs, W1, b1, W2, b2, W3, b3, table):
    tableP = _transpose_table(jnp.swapaxes(table, 0, 1))
    mlp_out = _mlp(numerical_input, W1, b1, W2, b2, W3, b3)
    fidx = _idx_prep(categorical_inputs.reshape(IDXR, GU))
    return _sc_gather(fidx, mlp_out, tableP)
